# baseline (speedup 1.0000x reference)
import functools

import jax
import jax.numpy as jnp
from jax import lax
from jax.experimental import pallas as pl
from jax.experimental.pallas import tpu as pltpu
from jax.experimental.pallas import tpu_sc as plsc

_DNUMS = lax.GatherDimensionNumbers(
    offset_dims=(), collapsed_slice_dims=(0,), start_index_map=(0,))


def _shuffle(vec, idx):
    return lax.gather(vec, idx.reshape(idx.shape[0], 1), _DNUMS, (1,),
                      mode=lax.GatherScatterMode.PROMISE_IN_BOUNDS)


NC = 2
NS = 16
NW = NC * NS
GRP = 16
NB = 3
LANES = 16


def _hsum(vec, lane):
    for sh in (8, 4, 2, 1):
        vec = vec + _shuffle(vec, jnp.bitwise_xor(lane, sh))
    return vec


@functools.lru_cache(maxsize=None)
def _make_sc(B, D, H):
    assert B % NW == 0
    rpw = B // NW
    ng = rpw // GRP
    dc = D // LANES
    hc = H // LANES
    vrt = D // NS
    assert vrt % 32 == 0

    mesh = plsc.VectorSubcoreMesh(core_axis_name="c", subcore_axis_name="s",
                                  num_cores=NC, num_subcores=NS)

    def body(x_hbm, t1_hbm, t2_hbm, w1a_hbm, w1b_hbm, w2a_hbm, w2b_hbm,
             b1a_hbm, b1b_hbm, b2a_hbm, b2b_hbm, out_hbm,
             idx_v, v_v, r1_v, r2_v, out_v, wa_v, w1b_v, w2b_v,
             ba1_v, ba2_v, bs1_v, bs2_v, vstage_v, vsh,
             sem0, sem1, sem2):
        cid = lax.axis_index("c")
        sid = lax.axis_index("s")
        wid = sid * NC + cid
        base = wid * rpw
        lane = lax.iota(jnp.int32, LANES)

        pltpu.sync_copy(x_hbm.at[pl.ds(base, rpw)], idx_v)

        sems = (sem0, sem1, sem2)
        handles = [None] * NB

        def fire(g, b):
            iv = idx_v[pl.ds(g * GRP, GRP)]
            h1 = pltpu.async_copy(t1_hbm.at[iv], r1_v.at[b], sems[b])
            h2 = pltpu.async_copy(t2_hbm.at[iv], r2_v.at[b], sems[b])
            handles[b] = (h1, h2)

        for b in range(NB):
            fire(b, b)

        pltpu.sync_copy(w1b_hbm, w1b_v)
        pltpu.sync_copy(w2b_hbm, w2b_v)

        def rowdots(wref, wvec_ref):
            def jb(j, accs):
                o = pl.ds(pl.multiple_of(j * LANES, LANES), LANES)
                wv = wvec_ref[o]
                return tuple(accs[r] + wref[r, o] * wv for r in range(GRP))
            zero = jnp.zeros((LANES,), jnp.float32)
            accs = lax.fori_loop(0, hc, jb, (zero,) * GRP)
            vec = jnp.zeros((LANES,), jnp.float32)
            for r in range(GRP):
                vec = vec + jnp.where(lane == r, _hsum(accs[r], lane), 0.0)
            return vec

        vbase = sid * vrt
        npass = vrt // 32
        for p in range(npass):
            pltpu.sync_copy(w1a_hbm.at[pl.ds(vbase + p * 32, 32)], wa_v)
            v1vecs = [rowdots(wa_v.at[pl.ds(k * GRP, GRP)], w1b_v) for k in range(2)]
            pltpu.sync_copy(w2a_hbm.at[pl.ds(vbase + p * 32, 32)], wa_v)
            for k in range(2):
                v2vec = rowdots(wa_v.at[pl.ds(k * GRP, GRP)], w2b_v)
                o = pl.ds(p * 32 + k * GRP, GRP)
                vstage_v[0, o] = v1vecs[k]
                vstage_v[1, o] = v1vecs[k] + v2vec
        pltpu.sync_copy(vstage_v.at[0], vsh.at[0, pl.ds(vbase, vrt)])
        pltpu.sync_copy(vstage_v.at[1], vsh.at[1, pl.ds(vbase, vrt)])

        pltpu.sync_copy(b1a_hbm, ba1_v)
        pltpu.sync_copy(b2a_hbm, ba2_v)
        bs1_v[...] = jnp.zeros((LANES,), jnp.float32)
        bs2_v[...] = jnp.zeros((LANES,), jnp.float32)
        pltpu.sync_copy(b1b_hbm, bs1_v.at[pl.ds(0, 1)])
        pltpu.sync_copy(b2b_hbm, bs2_v.at[pl.ds(0, 1)])

        def cb(j, acc):
            o = pl.ds(pl.multiple_of(j * LANES, LANES), LANES)
            return acc + ba1_v[o] * w1b_v[o] + ba2_v[o] * w2b_v[o]

        cacc = lax.fori_loop(0, hc, cb, jnp.zeros((LANES,), jnp.float32))
        c_vec = _hsum(cacc + bs1_v[...] + bs2_v[...], lane)

        plsc.subcore_barrier()
        pltpu.sync_copy(vsh, v_v)

        def compute(g, b):
            def jbody(j, accs):
                o = pl.ds(pl.multiple_of(j * LANES, LANES), LANES)
                v1c = v_v[0, o]
                v12c = v_v[1, o]
                return tuple(
                    accs[r] + r1_v[b, r, o] * v1c + r2_v[b, r, o] * v12c
                    for r in range(GRP))

            zero = jnp.zeros((LANES,), jnp.float32)
            accs = lax.fori_loop(0, dc, jbody, (zero,) * GRP)
            outv = c_vec
            for r in range(GRP):
                outv = outv + jnp.where(lane == r, _hsum(accs[r], lane), 0.0)
            out_v[pl.ds(g * GRP, GRP)] = outv

        for g in range(ng):
            b = g % NB
            for h in handles[b]:
                h.wait()
            compute(g, b)
            if g + NB < ng:
                fire(g + NB, b)

        pltpu.sync_copy(out_v, out_hbm.at[pl.ds(base, rpw)])

    return pl.kernel(
        body,
        out_type=jax.ShapeDtypeStruct((B,), jnp.float32),
        mesh=mesh,
        scratch_types=[
            pltpu.VMEM((rpw,), jnp.int32),
            pltpu.VMEM((2, D), jnp.float32),
            pltpu.VMEM((NB, GRP, D), jnp.float32),
            pltpu.VMEM((NB, GRP, D), jnp.float32),
            pltpu.VMEM((rpw,), jnp.float32),
            pltpu.VMEM((32, H), jnp.float32),
            pltpu.VMEM((H,), jnp.float32),
            pltpu.VMEM((H,), jnp.float32),
            pltpu.VMEM((H,), jnp.float32),
            pltpu.VMEM((H,), jnp.float32),
            pltpu.VMEM((LANES,), jnp.float32),
            pltpu.VMEM((LANES,), jnp.float32),
            pltpu.VMEM((2, D // NS), jnp.float32),
            pltpu.VMEM_SHARED((2, D), jnp.float32),
            pltpu.SemaphoreType.DMA,
            pltpu.SemaphoreType.DMA,
            pltpu.SemaphoreType.DMA,
        ],
    )


def kernel(x, table_1, table_2, W1a, b1a, W1b, b1b, W2a, b2a, W2b, b2b):
    B = x.shape[0]
    D = table_1.shape[1]
    H = W1a.shape[1]
    out = _make_sc(B, D, H)(x, table_1, table_2,
                            W1a, W1b.reshape(H), W2a, W2b.reshape(H),
                            b1a, b1b, b2a, b2b)
    return out.reshape(B, 1)


# speedup vs baseline: 1.1167x; 1.1167x over previous
import functools

import jax
import jax.numpy as jnp
from jax import lax
from jax.experimental import pallas as pl
from jax.experimental.pallas import tpu as pltpu
from jax.experimental.pallas import tpu_sc as plsc

_DNUMS = lax.GatherDimensionNumbers(
    offset_dims=(), collapsed_slice_dims=(0,), start_index_map=(0,))


def _shuffle(vec, idx):
    return lax.gather(vec, idx.reshape(idx.shape[0], 1), _DNUMS, (1,),
                      mode=lax.GatherScatterMode.PROMISE_IN_BOUNDS)


NC = 2
NS = 16
NW = NC * NS
GRP = 16
NB = 3
LANES = 16


def _fold_body(W1a_ref, W1b_ref, W2a_ref, W2b_ref,
               b1a_ref, b1b_ref, b2a_ref, b2b_ref, v_ref, c_ref):
    dn = (((0,), (1,)), ((), ()))
    v1 = lax.dot_general(W1b_ref[...], W1a_ref[...], dn,
                         preferred_element_type=jnp.float32)
    v2 = lax.dot_general(W2b_ref[...], W2a_ref[...], dn,
                         preferred_element_type=jnp.float32)
    v_ref[...] = jnp.concatenate([v1, v1 + v2], axis=0)
    c = (jnp.dot(b1a_ref[...], W1b_ref[...])[0, 0] + b1b_ref[0, 0]
         + jnp.dot(b2a_ref[...], W2b_ref[...])[0, 0] + b2b_ref[0, 0])
    c_ref[...] = jnp.full((1, LANES), c, jnp.float32)


@functools.lru_cache(maxsize=None)
def _make_fold(D, H):
    return pl.pallas_call(
        _fold_body,
        out_shape=(
            jax.ShapeDtypeStruct((2, D), jnp.float32),
            jax.ShapeDtypeStruct((1, LANES), jnp.float32),
        ),
    )


@functools.lru_cache(maxsize=None)
def _make_sc(B, D):
    assert B % NW == 0
    rpw = B // NW
    ng = rpw // GRP
    dc = D // LANES

    mesh = plsc.VectorSubcoreMesh(core_axis_name="c", subcore_axis_name="s",
                                  num_cores=NC, num_subcores=NS)

    def body(x_hbm, t1_hbm, t2_hbm, v_hbm, c_hbm, out_hbm,
             idx_v, v_v, c_v, r1_v, r2_v, out_v, sem0, sem1, sem2):
        wid = lax.axis_index("s") * NC + lax.axis_index("c")
        base = wid * rpw
        pltpu.sync_copy(x_hbm.at[pl.ds(base, rpw)], idx_v)
        pltpu.sync_copy(v_hbm, v_v)
        pltpu.sync_copy(c_hbm, c_v)

        sems = (sem0, sem1, sem2)
        handles = [None] * NB

        def fire(g, b):
            iv = idx_v[pl.ds(g * GRP, GRP)]
            h1 = pltpu.async_copy(t1_hbm.at[iv], r1_v.at[b], sems[b])
            h2 = pltpu.async_copy(t2_hbm.at[iv], r2_v.at[b], sems[b])
            handles[b] = (h1, h2)

        def compute(g, b):
            def jbody(j, accs):
                o = pl.ds(pl.multiple_of(j * LANES, LANES), LANES)
                v1c = v_v[0, o]
                v12c = v_v[1, o]
                return tuple(
                    accs[r] + r1_v[b, r, o] * v1c + r2_v[b, r, o] * v12c
                    for r in range(GRP))

            zero = jnp.zeros((LANES,), jnp.float32)
            accs = lax.fori_loop(0, dc, jbody, (zero,) * GRP)
            lane = lax.iota(jnp.int32, LANES)
            outv = c_v[...]
            for r in range(GRP):
                t = accs[r]
                for sh in (8, 4, 2, 1):
                    t = t + _shuffle(t, jnp.bitwise_xor(lane, sh))
                outv = outv + jnp.where(lane == r, t, 0.0)
            out_v[pl.ds(g * GRP, GRP)] = outv

        for b in range(min(NB, ng)):
            fire(b, b)
        for g in range(ng):
            b = g % NB
            for h in handles[b]:
                h.wait()
            compute(g, b)
            if g + NB < ng:
                fire(g + NB, b)

        pltpu.sync_copy(out_v, out_hbm.at[pl.ds(base, rpw)])

    return pl.kernel(
        body,
        out_type=jax.ShapeDtypeStruct((B,), jnp.float32),
        mesh=mesh,
        scratch_types=[
            pltpu.VMEM((rpw,), jnp.int32),
            pltpu.VMEM((2, D), jnp.float32),
            pltpu.VMEM((LANES,), jnp.float32),
            pltpu.VMEM((NB, GRP, D), jnp.float32),
            pltpu.VMEM((NB, GRP, D), jnp.float32),
            pltpu.VMEM((rpw,), jnp.float32),
            pltpu.SemaphoreType.DMA,
            pltpu.SemaphoreType.DMA,
            pltpu.SemaphoreType.DMA,
        ],
    )


def kernel(x, table_1, table_2, W1a, b1a, W1b, b1b, W2a, b2a, W2b, b2b):
    B = x.shape[0]
    D = table_1.shape[1]
    H = W1a.shape[1]
    vt, c = _make_fold(D, H)(W1a, W1b, W2a, W2b,
                             b1a.reshape(1, H), b1b.reshape(1, 1),
                             b2a.reshape(1, H), b2b.reshape(1, 1))
    out = _make_sc(B, D)(x, table_1, table_2, vt, c.reshape(LANES))
    return out.reshape(B, 1)
